# baseline (speedup 1.0000x reference)
import functools

import jax
import jax.numpy as jnp
from jax import lax
from jax.experimental import pallas as pl
from jax.experimental.pallas import tpu as pltpu
from jax.experimental.pallas import tpu_sc as plsc

N = 10000
E = 160000
NC = 2
NS = 16
NW = NC * NS
LANES = 16
UNROLL = 4
EPW = 4992
EBUF = E - (NW - 1) * EPW
CHUNKS = EBUF // LANES
NPAD = 10240
PBUF = NPAD + LANES
SENT = NPAD


NPW = NPAD // NW


def _sc_segment_last(s0f, ei):
  mesh = plsc.VectorSubcoreMesh(core_axis_name="c", subcore_axis_name="s")

  @functools.partial(
      pl.kernel,
      out_type=(jax.ShapeDtypeStruct((NW * NPAD,), jnp.int32),
                jax.ShapeDtypeStruct((3 * NPAD,), jnp.float32)),
      mesh=mesh,
      compiler_params=pltpu.CompilerParams(needs_layout_passes=False),
      scratch_types=[
          pltpu.VMEM((3 * N,), jnp.float32),
          pltpu.VMEM((2, EBUF), jnp.int32),
          pltpu.VMEM((PBUF,), jnp.int32),
          pltpu.VMEM((EBUF,), jnp.int32),
          pltpu.VMEM((EBUF,), jnp.int32),
          pltpu.VMEM((3 * NPW,), jnp.float32),
          pltpu.SemaphoreType.DMA,
          pltpu.SemaphoreType.DMA,
      ],
  )
  def k(s0_hbm, ei_hbm, out_hbm, st_hbm, s0_v, ei_v, p_v,
        dsel_v, vsel_v, st_v, sem0, sem1):
    wid = lax.axis_index("s") * NC + lax.axis_index("c")
    base = jnp.minimum(wid * EPW, E - EBUF)
    cp0 = pltpu.async_copy(s0_hbm, s0_v, sem0)
    cp1 = pltpu.async_copy(ei_hbm.at[:, pl.ds(base, EBUF)], ei_v, sem1)

    lane = lax.iota(jnp.int32, LANES)
    minus1 = jnp.full((LANES,), -1, jnp.int32)
    is_last = lane == LANES - 1
    nxt = jnp.minimum(lane + 1, LANES - 1)

    def init_body(i, carry):
      off = i * (LANES * 8)
      for u in range(8):
        p_v[pl.ds(off + u * LANES, LANES)] = minus1
      return carry

    lax.fori_loop(0, PBUF // (LANES * 8), init_body, 0)
    for r in range(PBUF % (LANES * 8) // LANES):
      p_v[pl.ds(PBUF - (r + 1) * LANES, LANES)] = minus1

    cp0.wait()
    cp1.wait()

    n0 = wid * NPW
    for c in range(NPW // LANES):
      nn = jnp.minimum(n0 + c * LANES + lane, N - 1) * 3
      for t in range(3):
        st_v[pl.ds(t * NPW + c * LANES, LANES)] = plsc.load_gather(
            s0_v, [nn + t])
    for t in range(3):
      pltpu.sync_copy(st_v.at[pl.ds(t * NPW, NPW)],
                      st_hbm.at[pl.ds(t * NPAD + n0, NPW)])

    def front(off):
      s = ei_v[0, pl.ds(off, LANES)]
      d = ei_v[1, pl.ds(off, LANES)]
      s3 = s * 3
      g0 = plsc.load_gather(s0_v, [s3])
      g1 = plsc.load_gather(s0_v, [s3 + 1])
      g2 = plsc.load_gather(s0_v, [s3 + 2])
      bit = (jnp.maximum(g1, g2) > g0).astype(jnp.int32)
      packed = (base + off + lane) * 2 + bit
      key = d * LANES + lane
      ks, vs = plsc.sort_key_val(key, packed)
      dsort = lax.shift_right_logical(ks, 4)
      dnext = lax.gather(
          dsort, nxt[:, None],
          lax.GatherDimensionNumbers(
              offset_dims=(), collapsed_slice_dims=(0,),
              start_index_map=(0,)),
          slice_sizes=(1,),
          mode=lax.GatherScatterMode.PROMISE_IN_BOUNDS)
      wmask = jnp.logical_or(is_last, dsort != dnext)
      dsel_v[pl.ds(off, LANES)] = jnp.where(wmask, dsort, SENT)
      vsel_v[pl.ds(off, LANES)] = vs

    @plsc.parallel_loop(0, CHUNKS, unroll=4)
    def _(i):
      front(i * LANES)

    def scat(i, carry):
      off = i * (LANES * UNROLL)
      for u in range(UNROLL):
        o = off + u * LANES
        plsc.store_scatter(
            p_v, [dsel_v[pl.ds(o, LANES)]], vsel_v[pl.ds(o, LANES)])
      return carry

    lax.fori_loop(0, CHUNKS // UNROLL, scat, 0)

    pltpu.sync_copy(p_v.at[pl.ds(0, NPAD)], out_hbm.at[pl.ds(wid * NPAD, NPAD)])

  return k(s0f, ei)


def _tc_update(pflat, stf, Tm):
  def body(p_ref, st_ref, tm_ref, out_ref):
    acc = p_ref[pl.ds(0, NPAD)]
    for w in range(1, NW):
      acc = jnp.maximum(acc, p_ref[pl.ds(w * NPAD, NPAD)])
    packed = jnp.reshape(acc, (1, NPAD))
    b = jnp.logical_and(packed >= 0, lax.bitwise_and(packed, 1) == 1)
    node = lax.broadcasted_iota(jnp.int32, (1, NPAD), 1)
    odd = lax.bitwise_and(node, 1) == 1
    t = tm_ref[...]
    m = jnp.max(t, axis=1, keepdims=True)
    e = jnp.exp(t - m)
    sm = e / jnp.sum(e, axis=1, keepdims=True)
    st = jnp.concatenate(
        [jnp.reshape(st_ref[pl.ds(s * NPAD, NPAD)], (1, NPAD))
         for s in range(3)], axis=0)
    dn = (((0,), (0,)), ((), ()))
    hi = lax.Precision.HIGHEST
    a0 = lax.dot_general(sm[0:3, :], st, dn, precision=hi)
    a1 = lax.dot_general(sm[3:6, :], st, dn, precision=hi)
    a3 = lax.dot_general(sm[9:12, :], st, dn, precision=hi)
    res = jnp.where(b, jnp.where(odd, a1, a3), a0)
    out_ref[...] = res[:, 0:N]

  return pl.pallas_call(
      body,
      out_shape=jax.ShapeDtypeStruct((3, N), jnp.float32),
  )(pflat, stf, Tm)


def kernel(s0, edge_index, T):
  pflat, stf = _sc_segment_last(s0.reshape(3 * N), edge_index)
  res = _tc_update(pflat, stf, T.reshape(27, 3))
  return res.T
